# Initial kernel scaffold; baseline (speedup 1.0000x reference)
import jax
import jax.numpy as jnp
from jax.experimental import pallas as pl


def kernel(frame, _label):
    raise NotImplementedError("write your pallas kernel here")


# speedup vs baseline: 9.5687x; 9.5687x over previous
import functools

import jax
import jax.numpy as jnp
from jax.experimental import pallas as pl

_T = 32768
_K = _T // 16 + 1
_ROWS = 128
_BLOCK_R = 8
_GRID = _ROWS // _BLOCK_R
_HI_BITS = 0x3F800000
_SEARCH_ITERS = 30


def _topk_bce_body(frame_ref, label_ref, out_ref):
    x = frame_ref[...]
    xi = jax.lax.bitcast_convert_type(x, jnp.int32)

    lo = jnp.zeros((_BLOCK_R, 1), jnp.int32)
    hi = jnp.full((_BLOCK_R, 1), _HI_BITS, jnp.int32)

    def step(_, carry):
        lo, hi = carry
        mid = (lo + hi) >> 1
        cnt = jnp.sum((xi >= mid).astype(jnp.int32), axis=1, keepdims=True)
        pred = cnt >= _K
        return jnp.where(pred, mid, lo), jnp.where(pred, hi, mid)

    lo, hi = jax.lax.fori_loop(0, _SEARCH_ITERS, step, (lo, hi))
    thr_bits = lo
    thr = jax.lax.bitcast_convert_type(thr_bits, jnp.float32)

    gt = xi > thr_bits
    cnt_gt = jnp.sum(gt.astype(jnp.float32), axis=1, keepdims=True)
    sum_gt = jnp.sum(jnp.where(gt, x, 0.0), axis=1, keepdims=True)
    anomaly = (sum_gt + (_K - cnt_gt) * thr) * (1.0 / _K)

    logp = jnp.maximum(jnp.log(anomaly), -100.0)
    log1mp = jnp.maximum(jnp.log(1.0 - anomaly), -100.0)
    lab = label_ref[...]
    contrib = jnp.sum(lab * logp + (1.0 - lab) * log1mp)

    i = pl.program_id(0)

    @pl.when(i == 0)
    def _init():
        out_ref[...] = jnp.zeros_like(out_ref)

    out_ref[...] += jnp.full((1, 1), contrib, jnp.float32)

    @pl.when(i == _GRID - 1)
    def _fini():
        out_ref[...] = out_ref[...] * (-1.0 / _ROWS)


@jax.jit
def kernel(frame, _label):
    label = _label.astype(jnp.float32).reshape(_ROWS, 1)
    out = pl.pallas_call(
        _topk_bce_body,
        grid=(_GRID,),
        in_specs=[
            pl.BlockSpec((_BLOCK_R, _T), lambda i: (i, 0)),
            pl.BlockSpec((_BLOCK_R, 1), lambda i: (i, 0)),
        ],
        out_specs=pl.BlockSpec((1, 1), lambda i: (0, 0)),
        out_shape=jax.ShapeDtypeStruct((1, 1), jnp.float32),
    )(frame, label)
    return out[0, 0]


# speedup vs baseline: 12.6724x; 1.3244x over previous
import functools

import jax
import jax.numpy as jnp
from jax.experimental import pallas as pl

_T = 32768
_K = _T // 16 + 1
_ROWS = 128
_BLOCK_R = 8
_GRID = _ROWS // _BLOCK_R
_HI_BITS = 0x3F800000
_SEARCH_ITERS = 30


def _tree_lane_sum(t):
    while t.shape[1] > 1:
        h = t.shape[1] // 2
        t = t[:, :h, :] + t[:, h:, :]
    return jnp.sum(t[:, 0, :], axis=-1, keepdims=True)


def _topk_bce_body(frame_ref, label_ref, out_ref):
    x = frame_ref[...]
    xi = jax.lax.bitcast_convert_type(x, jnp.int32)
    xi3 = xi.reshape(_BLOCK_R, _T // 128, 128)
    x3 = x.reshape(_BLOCK_R, _T // 128, 128)

    lo = jnp.zeros((_BLOCK_R, 1), jnp.int32)
    hi = jnp.full((_BLOCK_R, 1), _HI_BITS, jnp.int32)

    def step(_, carry):
        lo, hi = carry
        mid = ((lo + hi) >> 1)[:, :, None]
        cnt = _tree_lane_sum((xi3 >= mid).astype(jnp.int32))
        pred = cnt >= _K
        mid = mid[:, :, 0]
        return jnp.where(pred, mid, lo), jnp.where(pred, hi, mid)

    lo, hi = jax.lax.fori_loop(0, _SEARCH_ITERS, step, (lo, hi))
    thr_bits = lo
    thr = jax.lax.bitcast_convert_type(thr_bits, jnp.float32)

    gt = xi3 > thr_bits[:, :, None]
    cnt_gt = _tree_lane_sum(gt.astype(jnp.float32))
    sum_gt = _tree_lane_sum(jnp.where(gt, x3, 0.0))
    anomaly = (sum_gt + (_K - cnt_gt) * thr) * (1.0 / _K)

    logp = jnp.maximum(jnp.log(anomaly), -100.0)
    log1mp = jnp.maximum(jnp.log(1.0 - anomaly), -100.0)
    lab = label_ref[...]
    contrib = jnp.sum(lab * logp + (1.0 - lab) * log1mp)

    i = pl.program_id(0)

    @pl.when(i == 0)
    def _init():
        out_ref[...] = jnp.zeros_like(out_ref)

    out_ref[...] += jnp.full((1, 1), contrib, jnp.float32)

    @pl.when(i == _GRID - 1)
    def _fini():
        out_ref[...] = out_ref[...] * (-1.0 / _ROWS)


@jax.jit
def kernel(frame, _label):
    label = _label.astype(jnp.float32).reshape(_ROWS, 1)
    out = pl.pallas_call(
        _topk_bce_body,
        grid=(_GRID,),
        in_specs=[
            pl.BlockSpec((_BLOCK_R, _T), lambda i: (i, 0)),
            pl.BlockSpec((_BLOCK_R, 1), lambda i: (i, 0)),
        ],
        out_specs=pl.BlockSpec((1, 1), lambda i: (0, 0)),
        out_shape=jax.ShapeDtypeStruct((1, 1), jnp.float32),
    )(frame, label)
    return out[0, 0]


# speedup vs baseline: 16.7682x; 1.3232x over previous
import functools

import jax
import jax.numpy as jnp
from jax.experimental import pallas as pl

_T = 32768
_K = _T // 16 + 1
_ROWS = 128
_BLOCK_R = 8
_GRID = _ROWS // _BLOCK_R
_HI_BITS = 0x3F800000
_SEARCH_ITERS = 30


_W_ACC = 2048


def _lane_fold(acc):
    w = acc.shape[1]
    while w > 128:
        w //= 2
        acc = acc[:, :w] + acc[:, w:]
    return jnp.sum(acc, axis=-1, keepdims=True)


def _topk_bce_body(frame_ref, label_ref, out_ref):
    x = frame_ref[...]
    xi = jax.lax.bitcast_convert_type(x, jnp.int32)

    lo = jnp.zeros((_BLOCK_R, 1), jnp.int32)
    hi = jnp.full((_BLOCK_R, 1), _HI_BITS, jnp.int32)

    def step(_, carry):
        lo, hi = carry
        mid = (lo + hi) >> 1
        acc = jnp.zeros((_BLOCK_R, _W_ACC), jnp.int32)
        for j in range(_T // _W_ACC):
            sl = xi[:, j * _W_ACC:(j + 1) * _W_ACC]
            acc = acc + (sl >= mid).astype(jnp.int32)
        cnt = _lane_fold(acc)
        pred = cnt >= _K
        return jnp.where(pred, mid, lo), jnp.where(pred, hi, mid)

    lo, hi = jax.lax.fori_loop(0, _SEARCH_ITERS, step, (lo, hi))
    thr_bits = lo
    thr = jax.lax.bitcast_convert_type(thr_bits, jnp.float32)

    acc_c = jnp.zeros((_BLOCK_R, _W_ACC), jnp.float32)
    acc_s = jnp.zeros((_BLOCK_R, _W_ACC), jnp.float32)
    for j in range(_T // _W_ACC):
        sl_i = xi[:, j * _W_ACC:(j + 1) * _W_ACC]
        sl_x = x[:, j * _W_ACC:(j + 1) * _W_ACC]
        gt = sl_i > thr_bits
        acc_c = acc_c + gt.astype(jnp.float32)
        acc_s = acc_s + jnp.where(gt, sl_x, 0.0)
    cnt_gt = _lane_fold(acc_c)
    sum_gt = _lane_fold(acc_s)
    anomaly = (sum_gt + (_K - cnt_gt) * thr) * (1.0 / _K)

    logp = jnp.maximum(jnp.log(anomaly), -100.0)
    log1mp = jnp.maximum(jnp.log(1.0 - anomaly), -100.0)
    lab = label_ref[...]
    contrib = jnp.sum(lab * logp + (1.0 - lab) * log1mp)

    i = pl.program_id(0)

    @pl.when(i == 0)
    def _init():
        out_ref[...] = jnp.zeros_like(out_ref)

    out_ref[...] += jnp.full((1, 1), contrib, jnp.float32)

    @pl.when(i == _GRID - 1)
    def _fini():
        out_ref[...] = out_ref[...] * (-1.0 / _ROWS)


@jax.jit
def kernel(frame, _label):
    label = _label.astype(jnp.float32).reshape(_ROWS, 1)
    out = pl.pallas_call(
        _topk_bce_body,
        grid=(_GRID,),
        in_specs=[
            pl.BlockSpec((_BLOCK_R, _T), lambda i: (i, 0)),
            pl.BlockSpec((_BLOCK_R, 1), lambda i: (i, 0)),
        ],
        out_specs=pl.BlockSpec((1, 1), lambda i: (0, 0)),
        out_shape=jax.ShapeDtypeStruct((1, 1), jnp.float32),
    )(frame, label)
    return out[0, 0]


# speedup vs baseline: 24.7861x; 1.4782x over previous
import functools

import jax
import jax.numpy as jnp
from jax.experimental import pallas as pl

_T = 32768
_K = _T // 16 + 1
_ROWS = 128
_BLOCK_R = 32
_GRID = _ROWS // _BLOCK_R
_HI_BITS = 0x3F800000
_SEARCH_ITERS = 30


_W_ACC = 2048


def _lane_fold(acc):
    w = acc.shape[1]
    while w > 128:
        w //= 2
        acc = acc[:, :w] + acc[:, w:]
    return jnp.sum(acc, axis=-1, keepdims=True)


def _topk_bce_body(frame_ref, label_ref, out_ref):
    x = frame_ref[...]
    xi = jax.lax.bitcast_convert_type(x, jnp.int32)

    lo = jnp.zeros((_BLOCK_R, 1), jnp.int32)
    hi = jnp.full((_BLOCK_R, 1), _HI_BITS, jnp.int32)

    def step(_, carry):
        lo, hi = carry
        mid = (lo + hi) >> 1
        acc = jnp.zeros((_BLOCK_R, _W_ACC), jnp.int32)
        for j in range(_T // _W_ACC):
            sl = xi[:, j * _W_ACC:(j + 1) * _W_ACC]
            acc = acc + (sl >= mid).astype(jnp.int32)
        cnt = _lane_fold(acc)
        pred = cnt >= _K
        return jnp.where(pred, mid, lo), jnp.where(pred, hi, mid)

    lo, hi = jax.lax.fori_loop(0, _SEARCH_ITERS, step, (lo, hi))
    thr_bits = lo
    thr = jax.lax.bitcast_convert_type(thr_bits, jnp.float32)

    acc_c = jnp.zeros((_BLOCK_R, _W_ACC), jnp.float32)
    acc_s = jnp.zeros((_BLOCK_R, _W_ACC), jnp.float32)
    for j in range(_T // _W_ACC):
        sl_i = xi[:, j * _W_ACC:(j + 1) * _W_ACC]
        sl_x = x[:, j * _W_ACC:(j + 1) * _W_ACC]
        gt = sl_i > thr_bits
        acc_c = acc_c + gt.astype(jnp.float32)
        acc_s = acc_s + jnp.where(gt, sl_x, 0.0)
    cnt_gt = _lane_fold(acc_c)
    sum_gt = _lane_fold(acc_s)
    anomaly = (sum_gt + (_K - cnt_gt) * thr) * (1.0 / _K)

    logp = jnp.maximum(jnp.log(anomaly), -100.0)
    log1mp = jnp.maximum(jnp.log(1.0 - anomaly), -100.0)
    lab = label_ref[...]
    contrib = jnp.sum(lab * logp + (1.0 - lab) * log1mp)

    i = pl.program_id(0)

    @pl.when(i == 0)
    def _init():
        out_ref[...] = jnp.zeros_like(out_ref)

    out_ref[...] += jnp.full((1, 1), contrib, jnp.float32)

    @pl.when(i == _GRID - 1)
    def _fini():
        out_ref[...] = out_ref[...] * (-1.0 / _ROWS)


@jax.jit
def kernel(frame, _label):
    label = _label.astype(jnp.float32).reshape(_ROWS, 1)
    out = pl.pallas_call(
        _topk_bce_body,
        grid=(_GRID,),
        in_specs=[
            pl.BlockSpec((_BLOCK_R, _T), lambda i: (i, 0)),
            pl.BlockSpec((_BLOCK_R, 1), lambda i: (i, 0)),
        ],
        out_specs=pl.BlockSpec((1, 1), lambda i: (0, 0)),
        out_shape=jax.ShapeDtypeStruct((1, 1), jnp.float32),
    )(frame, label)
    return out[0, 0]


# speedup vs baseline: 26.4374x; 1.0666x over previous
import functools

import jax
import jax.numpy as jnp
from jax.experimental import pallas as pl

_T = 32768
_K = _T // 16 + 1
_ROWS = 128
_BLOCK_R = 64
_GRID = _ROWS // _BLOCK_R
_HI_BITS = 0x3F800000
_SEARCH_ITERS = 30


_W_ACC = 2048


def _lane_fold(acc):
    w = acc.shape[1]
    while w > 128:
        w //= 2
        acc = acc[:, :w] + acc[:, w:]
    return jnp.sum(acc, axis=-1, keepdims=True)


def _topk_bce_body(frame_ref, label_ref, out_ref):
    x = frame_ref[...]
    xi = jax.lax.bitcast_convert_type(x, jnp.int32)

    lo = jnp.zeros((_BLOCK_R, 1), jnp.int32)
    hi = jnp.full((_BLOCK_R, 1), _HI_BITS, jnp.int32)

    def step(_, carry):
        lo, hi = carry
        mid = (lo + hi) >> 1
        acc = jnp.zeros((_BLOCK_R, _W_ACC), jnp.int32)
        for j in range(_T // _W_ACC):
            sl = xi[:, j * _W_ACC:(j + 1) * _W_ACC]
            acc = acc + (sl >= mid).astype(jnp.int32)
        cnt = _lane_fold(acc)
        pred = cnt >= _K
        return jnp.where(pred, mid, lo), jnp.where(pred, hi, mid)

    lo, hi = jax.lax.fori_loop(0, _SEARCH_ITERS, step, (lo, hi))
    thr_bits = lo
    thr = jax.lax.bitcast_convert_type(thr_bits, jnp.float32)

    acc_c = jnp.zeros((_BLOCK_R, _W_ACC), jnp.float32)
    acc_s = jnp.zeros((_BLOCK_R, _W_ACC), jnp.float32)
    for j in range(_T // _W_ACC):
        sl_i = xi[:, j * _W_ACC:(j + 1) * _W_ACC]
        sl_x = x[:, j * _W_ACC:(j + 1) * _W_ACC]
        gt = sl_i > thr_bits
        acc_c = acc_c + gt.astype(jnp.float32)
        acc_s = acc_s + jnp.where(gt, sl_x, 0.0)
    cnt_gt = _lane_fold(acc_c)
    sum_gt = _lane_fold(acc_s)
    anomaly = (sum_gt + (_K - cnt_gt) * thr) * (1.0 / _K)

    logp = jnp.maximum(jnp.log(anomaly), -100.0)
    log1mp = jnp.maximum(jnp.log(1.0 - anomaly), -100.0)
    lab = label_ref[...]
    contrib = jnp.sum(lab * logp + (1.0 - lab) * log1mp)

    i = pl.program_id(0)

    @pl.when(i == 0)
    def _init():
        out_ref[...] = jnp.zeros_like(out_ref)

    out_ref[...] += jnp.full((1, 1), contrib, jnp.float32)

    @pl.when(i == _GRID - 1)
    def _fini():
        out_ref[...] = out_ref[...] * (-1.0 / _ROWS)


@jax.jit
def kernel(frame, _label):
    label = _label.astype(jnp.float32).reshape(_ROWS, 1)
    out = pl.pallas_call(
        _topk_bce_body,
        grid=(_GRID,),
        in_specs=[
            pl.BlockSpec((_BLOCK_R, _T), lambda i: (i, 0)),
            pl.BlockSpec((_BLOCK_R, 1), lambda i: (i, 0)),
        ],
        out_specs=pl.BlockSpec((1, 1), lambda i: (0, 0)),
        out_shape=jax.ShapeDtypeStruct((1, 1), jnp.float32),
    )(frame, label)
    return out[0, 0]
